# baseline (speedup 1.0000x reference)
import functools

import numpy as np

import jax
import jax.numpy as jnp
from jax import lax
from jax.experimental import pallas as pl
from jax.experimental.pallas import tpu as pltpu

_BN_EPS = 1e-3
_VMEM_LIMIT = 48 * 1024 * 1024
_G = 2


def _round_up(x, m):
    return ((x + m - 1) // m) * m


def _fold_bn(p):
    scale = p["gamma"] * lax.rsqrt(p["var"] + _BN_EPS)
    bias = p["beta"] - p["mean"] * scale
    return scale, bias


def _pw_weight(p):
    scale, bias = _fold_bn(p)
    return p["w"][:, :, 0, 0] * scale[:, None], bias


def _tap_weights_all(ps_axes):
    gamma = jnp.stack([p["gamma"] for p, _ in ps_axes])
    var = jnp.stack([p["var"] for p, _ in ps_axes])
    beta = jnp.stack([p["beta"] for p, _ in ps_axes])
    mean = jnp.stack([p["mean"] for p, _ in ps_axes])
    scale = gamma * lax.rsqrt(var + _BN_EPS)
    bias = beta - mean * scale
    w = jnp.stack([p["w"][:, :, 0, :] if ax == "w" else p["w"][:, :, :, 0]
                   for p, ax in ps_axes])
    w = jnp.transpose(w, (0, 3, 1, 2)) * scale[:, None, :, None]
    return w.reshape(-1, w.shape[-1]), bias.reshape(-1, 1)


def _offset_masks(h, w, lp, offsets, g):
    l = np.arange(lp)
    hh, ww = l // w, l % w
    valid = l < h * w
    rows = [valid & (hh + dh >= 0) & (hh + dh < h) & (ww + dw >= 0) & (ww + dw < w)
            for dh, dw in offsets]
    m = np.stack(rows).astype(np.float32)
    return jnp.asarray(np.tile(m, (1, g)))


def _conv7(h, w, b, m_ref, unit, n2):
    co = w.shape[0] // 7
    z = jnp.dot(w, h, preferred_element_type=jnp.float32)
    acc = None
    for k in range(7):
        part = z[k * co:(k + 1) * co]
        shift = ((3 - k) * unit) % n2
        if shift:
            part = pltpu.roll(part, shift=shift, axis=1)
        term = part * m_ref[k:k + 1, :]
        acc = term if acc is None else acc + term
    return jnp.maximum(acc + b, 0.0)


def _fused_kernel(w0_ref, b0_ref, pb_ref, mw_ref, mh_ref, mv_ref, mp_ref,
                  wt_ref, bt_ref,
                  x_ref, o_ref, xs_ref, *, c7, w_stride, l, lp, g):
    n2 = g * lp
    bf = jnp.bfloat16
    for i in range(g):
        xs_ref[:, i * lp:i * lp + l] = x_ref[i].astype(bf)
        if l < lp:
            xs_ref[:, i * lp + l:(i + 1) * lp] = jnp.zeros(
                (x_ref.shape[1], lp - l), bf)
    x2 = xs_ref[...]

    y0 = jnp.dot(w0_ref[...], x2, preferred_element_type=jnp.float32)
    y0 = y0 + b0_ref[...]
    h7 = jnp.maximum(y0[:c7], 0.0).astype(bf)
    hd = jnp.maximum(y0[c7:2 * c7], 0.0).astype(bf)
    b1 = jnp.maximum(y0[2 * c7:2 * c7 + 192], 0.0)
    pz = y0[2 * c7 + 192:]

    def st(h, s, m_ref, unit):
        return _conv7(h, wt_ref[s * 7 * c7:(s + 1) * 7 * c7],
                      bt_ref[s * c7:(s + 1) * c7], m_ref, unit, n2)

    t = st(h7, 0, mw_ref, 1).astype(bf)
    b7 = st(t, 1, mh_ref, w_stride)

    t = st(hd, 2, mh_ref, w_stride).astype(bf)
    t = st(t, 3, mw_ref, 1).astype(bf)
    t = st(t, 4, mh_ref, w_stride).astype(bf)
    bd = st(t, 5, mw_ref, 1)

    pv = None
    for i, dh in enumerate((-1, 0, 1)):
        shift = (-dh * w_stride) % n2
        part = pltpu.roll(pz, shift=shift, axis=1) if shift else pz
        term = part * mv_ref[i:i + 1, :]
        pv = term if pv is None else pv + term
    pp = None
    for i, dw in enumerate((-1, 0, 1)):
        shift = (-dw) % n2
        part = pltpu.roll(pv, shift=shift, axis=1) if shift else pv
        term = part * mp_ref[i:i + 1, :]
        pp = term if pp is None else pp + term
    bp = jnp.maximum(pp + pb_ref[...], 0.0)

    out = jnp.concatenate([b1, b7, bd, bp], axis=0)
    for i in range(g):
        o_ref[i] = out[:, i * lp:i * lp + l].astype(o_ref.dtype)


@jax.jit
def _forward(x, params):
    n, c_in, h, w = x.shape
    c7 = params["branch7x7_1"]["w"].shape[0]
    l = h * w
    lp = _round_up(l, 128)
    g = _G if n % _G == 0 else 1
    n2 = g * lp
    bf = jnp.bfloat16

    xf = x.reshape(n, c_in, l)

    pw_names = ["branch7x7_1", "branch7x7dbl_1", "branch1x1", "branch_pool"]
    pw = [params[nm] for nm in pw_names]
    gamma = jnp.stack([p["gamma"] for p in pw])
    var = jnp.stack([p["var"] for p in pw])
    beta = jnp.stack([p["beta"] for p in pw])
    mean = jnp.stack([p["mean"] for p in pw])
    scale = gamma * lax.rsqrt(var + _BN_EPS)
    bias4 = beta - mean * scale
    scale = scale.at[3].divide(9.0)
    w0 = (jnp.stack([p["w"][:, :, 0, 0] for p in pw])
          * scale[:, :, None]).reshape(-1, c_in).astype(bf)
    bias_p = bias4[3]
    b0 = jnp.concatenate(
        [bias4[0], bias4[1], bias4[2], jnp.zeros_like(bias_p)])[:, None]

    wt, bt = _tap_weights_all([
        (params["branch7x7_2"], "w"), (params["branch7x7_3"], "h"),
        (params["branch7x7dbl_2"], "h"), (params["branch7x7dbl_3"], "w"),
        (params["branch7x7dbl_4"], "h"), (params["branch7x7dbl_5"], "w")])

    mask_w = _offset_masks(h, w, lp, [(0, k - 3) for k in range(7)], g)
    mask_h = _offset_masks(h, w, lp, [(k - 3, 0) for k in range(7)], g)
    mask_v = _offset_masks(h, w, lp, [(dh, 0) for dh in (-1, 0, 1)], g)
    mask_p = _offset_masks(h, w, lp, [(0, dw) for dw in (-1, 0, 1)], g)

    m_tot = w0.shape[0]
    c_out = 4 * 192

    def bcast(shape):
        return pl.BlockSpec(shape, lambda i: tuple(0 for _ in shape))

    out = pl.pallas_call(
        functools.partial(_fused_kernel, c7=c7, w_stride=w, l=l, lp=lp, g=g),
        out_shape=jax.ShapeDtypeStruct((n, c_out, l), x.dtype),
        grid=(n // g,),
        in_specs=[
            bcast((m_tot, c_in)),
            bcast((m_tot, 1)),
            bcast((192, 1)),
            bcast((7, n2)),
            bcast((7, n2)),
            bcast((3, n2)),
            bcast((3, n2)),
            bcast((6 * 7 * c7, c7)), bcast((6 * c7, 1)),
            pl.BlockSpec((g, c_in, l), lambda i: (i, 0, 0)),
        ],
        out_specs=pl.BlockSpec((g, c_out, l), lambda i: (i, 0, 0)),
        scratch_shapes=[pltpu.VMEM((c_in, g * lp), bf)],
        compiler_params=pltpu.CompilerParams(
            dimension_semantics=("parallel",),
            vmem_limit_bytes=_VMEM_LIMIT),
    )(w0, b0, bias_p[:, None],
      mask_w, mask_h, mask_v, mask_p,
      wt.astype(bf), bt, xf)

    return out.reshape(n, c_out, h, w)


def kernel(x, branch1x1__w, branch1x1__gamma, branch1x1__beta, branch1x1__mean, branch1x1__var, branch7x7_1__w, branch7x7_1__gamma, branch7x7_1__beta, branch7x7_1__mean, branch7x7_1__var, branch7x7_2__w, branch7x7_2__gamma, branch7x7_2__beta, branch7x7_2__mean, branch7x7_2__var, branch7x7_3__w, branch7x7_3__gamma, branch7x7_3__beta, branch7x7_3__mean, branch7x7_3__var, branch7x7dbl_1__w, branch7x7dbl_1__gamma, branch7x7dbl_1__beta, branch7x7dbl_1__mean, branch7x7dbl_1__var, branch7x7dbl_2__w, branch7x7dbl_2__gamma, branch7x7dbl_2__beta, branch7x7dbl_2__mean, branch7x7dbl_2__var, branch7x7dbl_3__w, branch7x7dbl_3__gamma, branch7x7dbl_3__beta, branch7x7dbl_3__mean, branch7x7dbl_3__var, branch7x7dbl_4__w, branch7x7dbl_4__gamma, branch7x7dbl_4__beta, branch7x7dbl_4__mean, branch7x7dbl_4__var, branch7x7dbl_5__w, branch7x7dbl_5__gamma, branch7x7dbl_5__beta, branch7x7dbl_5__mean, branch7x7dbl_5__var, branch_pool__w, branch_pool__gamma, branch_pool__beta, branch_pool__mean, branch_pool__var):
    params = {
        "branch1x1": {"w": branch1x1__w, "gamma": branch1x1__gamma, "beta": branch1x1__beta, "mean": branch1x1__mean, "var": branch1x1__var},
        "branch7x7_1": {"w": branch7x7_1__w, "gamma": branch7x7_1__gamma, "beta": branch7x7_1__beta, "mean": branch7x7_1__mean, "var": branch7x7_1__var},
        "branch7x7_2": {"w": branch7x7_2__w, "gamma": branch7x7_2__gamma, "beta": branch7x7_2__beta, "mean": branch7x7_2__mean, "var": branch7x7_2__var},
        "branch7x7_3": {"w": branch7x7_3__w, "gamma": branch7x7_3__gamma, "beta": branch7x7_3__beta, "mean": branch7x7_3__mean, "var": branch7x7_3__var},
        "branch7x7dbl_1": {"w": branch7x7dbl_1__w, "gamma": branch7x7dbl_1__gamma, "beta": branch7x7dbl_1__beta, "mean": branch7x7dbl_1__mean, "var": branch7x7dbl_1__var},
        "branch7x7dbl_2": {"w": branch7x7dbl_2__w, "gamma": branch7x7dbl_2__gamma, "beta": branch7x7dbl_2__beta, "mean": branch7x7dbl_2__mean, "var": branch7x7dbl_2__var},
        "branch7x7dbl_3": {"w": branch7x7dbl_3__w, "gamma": branch7x7dbl_3__gamma, "beta": branch7x7dbl_3__beta, "mean": branch7x7dbl_3__mean, "var": branch7x7dbl_3__var},
        "branch7x7dbl_4": {"w": branch7x7dbl_4__w, "gamma": branch7x7dbl_4__gamma, "beta": branch7x7dbl_4__beta, "mean": branch7x7dbl_4__mean, "var": branch7x7dbl_4__var},
        "branch7x7dbl_5": {"w": branch7x7dbl_5__w, "gamma": branch7x7dbl_5__gamma, "beta": branch7x7dbl_5__beta, "mean": branch7x7dbl_5__mean, "var": branch7x7dbl_5__var},
        "branch_pool": {"w": branch_pool__w, "gamma": branch_pool__gamma, "beta": branch_pool__beta, "mean": branch_pool__mean, "var": branch_pool__var},
    }
    return _forward(x, params)


# speedup vs baseline: 1.1444x; 1.1444x over previous
import functools

import numpy as np

import jax
import jax.numpy as jnp
from jax import lax
from jax.experimental import pallas as pl
from jax.experimental.pallas import tpu as pltpu

_BN_EPS = 1e-3
_VMEM_LIMIT = 48 * 1024 * 1024
_G = 2


def _round_up(x, m):
    return ((x + m - 1) // m) * m


def _fold_bn(p):
    scale = p["gamma"] * lax.rsqrt(p["var"] + _BN_EPS)
    bias = p["beta"] - p["mean"] * scale
    return scale, bias


def _pw_weight(p):
    scale, bias = _fold_bn(p)
    return p["w"][:, :, 0, 0] * scale[:, None], bias


def _tap_weight(p, axis):
    scale, bias = _fold_bn(p)
    w = p["w"][:, :, 0, :] if axis == "w" else p["w"][:, :, :, 0]
    w = jnp.transpose(w, (2, 0, 1)) * scale[None, :, None]
    return w.reshape(-1, w.shape[-1]), bias


def _offset_masks(h, w, lp, offsets, g):
    l = np.arange(lp)
    hh, ww = l // w, l % w
    valid = l < h * w
    rows = [valid & (hh + dh >= 0) & (hh + dh < h) & (ww + dw >= 0) & (ww + dw < w)
            for dh, dw in offsets]
    m = np.stack(rows).astype(np.float32)
    return jnp.asarray(np.tile(m, (1, g)))


def _conv7(h, w, b, m_ref, unit, n2):
    co = w.shape[0] // 7
    z = jnp.dot(w, h, preferred_element_type=jnp.float32)
    acc = None
    for k in range(7):
        part = z[k * co:(k + 1) * co]
        shift = ((3 - k) * unit) % n2
        if shift:
            part = pltpu.roll(part, shift=shift, axis=1)
        term = part * m_ref[k:k + 1, :]
        acc = term if acc is None else acc + term
    return jnp.maximum(acc + b, 0.0)


def _fused_kernel(w0_ref, b0_ref, pb_ref, mw_ref, mh_ref, mv_ref, mp_ref,
                  w2_ref, b2_ref, w3_ref, b3_ref,
                  wd2_ref, bd2_ref, wd3_ref, bd3_ref,
                  wd4_ref, bd4_ref, wd5_ref, bd5_ref,
                  x_ref, o_ref, xs_ref, *, c7, w_stride, l, lp, g):
    n2 = g * lp
    bf = jnp.bfloat16
    for i in range(g):
        xs_ref[:, i * lp:i * lp + l] = x_ref[i].astype(bf)
        if l < lp:
            xs_ref[:, i * lp + l:(i + 1) * lp] = jnp.zeros(
                (x_ref.shape[1], lp - l), bf)
    x2 = xs_ref[...]

    y0 = jnp.dot(w0_ref[...], x2, preferred_element_type=jnp.float32)
    y0 = y0 + b0_ref[...]
    h7 = jnp.maximum(y0[:c7], 0.0).astype(bf)
    hd = jnp.maximum(y0[c7:2 * c7], 0.0).astype(bf)
    b1 = jnp.maximum(y0[2 * c7:2 * c7 + 192], 0.0)
    pz = y0[2 * c7 + 192:]

    t = _conv7(h7, w2_ref[...], b2_ref[...], mw_ref, 1, n2).astype(bf)
    b7 = _conv7(t, w3_ref[...], b3_ref[...], mh_ref, w_stride, n2)

    t = _conv7(hd, wd2_ref[...], bd2_ref[...], mh_ref, w_stride, n2).astype(bf)
    t = _conv7(t, wd3_ref[...], bd3_ref[...], mw_ref, 1, n2).astype(bf)
    t = _conv7(t, wd4_ref[...], bd4_ref[...], mh_ref, w_stride, n2).astype(bf)
    bd = _conv7(t, wd5_ref[...], bd5_ref[...], mw_ref, 1, n2)

    pv = None
    for i, dh in enumerate((-1, 0, 1)):
        shift = (-dh * w_stride) % n2
        part = pltpu.roll(pz, shift=shift, axis=1) if shift else pz
        term = part * mv_ref[i:i + 1, :]
        pv = term if pv is None else pv + term
    pp = None
    for i, dw in enumerate((-1, 0, 1)):
        shift = (-dw) % n2
        part = pltpu.roll(pv, shift=shift, axis=1) if shift else pv
        term = part * mp_ref[i:i + 1, :]
        pp = term if pp is None else pp + term
    bp = jnp.maximum(pp + pb_ref[...], 0.0)

    out = jnp.concatenate([b1, b7, bd, bp], axis=0)
    for i in range(g):
        o_ref[i] = out[:, i * lp:i * lp + l].astype(o_ref.dtype)


@jax.jit
def _forward(x, params):
    n, c_in, h, w = x.shape
    c7 = params["branch7x7_1"]["w"].shape[0]
    l = h * w
    lp = _round_up(l, 128)
    g = _G if n % _G == 0 else 1
    n2 = g * lp
    bf = jnp.bfloat16

    xf = x.reshape(n, c_in, l).astype(bf)

    w_h7, b_h7 = _pw_weight(params["branch7x7_1"])
    w_hd, b_hd = _pw_weight(params["branch7x7dbl_1"])
    w_b1, b_b1 = _pw_weight(params["branch1x1"])
    scale_p, bias_p = _fold_bn(params["branch_pool"])
    w_bp = params["branch_pool"]["w"][:, :, 0, 0] * (scale_p[:, None] / 9.0)
    w0 = jnp.concatenate([w_h7, w_hd, w_b1, w_bp], axis=0).astype(bf)
    b0 = jnp.concatenate(
        [b_h7, b_hd, b_b1, jnp.zeros_like(bias_p)], axis=0)[:, None]

    w2, b2 = _tap_weight(params["branch7x7_2"], "w")
    w3, b3 = _tap_weight(params["branch7x7_3"], "h")
    wd2, bd2 = _tap_weight(params["branch7x7dbl_2"], "h")
    wd3, bd3 = _tap_weight(params["branch7x7dbl_3"], "w")
    wd4, bd4 = _tap_weight(params["branch7x7dbl_4"], "h")
    wd5, bd5 = _tap_weight(params["branch7x7dbl_5"], "w")

    mask_w = _offset_masks(h, w, lp, [(0, k - 3) for k in range(7)], g)
    mask_h = _offset_masks(h, w, lp, [(k - 3, 0) for k in range(7)], g)
    mask_v = _offset_masks(h, w, lp, [(dh, 0) for dh in (-1, 0, 1)], g)
    mask_p = _offset_masks(h, w, lp, [(0, dw) for dw in (-1, 0, 1)], g)

    m_tot = w0.shape[0]
    c_out = 4 * 192

    def bcast(shape):
        return pl.BlockSpec(shape, lambda i: tuple(0 for _ in shape))

    out = pl.pallas_call(
        functools.partial(_fused_kernel, c7=c7, w_stride=w, l=l, lp=lp, g=g),
        out_shape=jax.ShapeDtypeStruct((n, c_out, l), bf),
        grid=(n // g,),
        in_specs=[
            bcast((m_tot, c_in)),
            bcast((m_tot, 1)),
            bcast((192, 1)),
            bcast((7, n2)),
            bcast((7, n2)),
            bcast((3, n2)),
            bcast((3, n2)),
            bcast((7 * c7, c7)), bcast((c7, 1)),
            bcast((7 * 192, c7)), bcast((192, 1)),
            bcast((7 * c7, c7)), bcast((c7, 1)),
            bcast((7 * c7, c7)), bcast((c7, 1)),
            bcast((7 * c7, c7)), bcast((c7, 1)),
            bcast((7 * 192, c7)), bcast((192, 1)),
            pl.BlockSpec((g, c_in, l), lambda i: (i, 0, 0)),
        ],
        out_specs=pl.BlockSpec((g, c_out, l), lambda i: (i, 0, 0)),
        scratch_shapes=[pltpu.VMEM((c_in, g * lp), bf)],
        compiler_params=pltpu.CompilerParams(
            dimension_semantics=("parallel",),
            vmem_limit_bytes=_VMEM_LIMIT),
    )(w0, b0, bias_p[:, None],
      mask_w, mask_h, mask_v, mask_p,
      w2.astype(bf), b2[:, None], w3.astype(bf), b3[:, None],
      wd2.astype(bf), bd2[:, None], wd3.astype(bf), bd3[:, None],
      wd4.astype(bf), bd4[:, None], wd5.astype(bf), bd5[:, None],
      xf)

    return out.astype(jnp.float32).reshape(n, c_out, h, w)


def kernel(x, branch1x1__w, branch1x1__gamma, branch1x1__beta, branch1x1__mean, branch1x1__var, branch7x7_1__w, branch7x7_1__gamma, branch7x7_1__beta, branch7x7_1__mean, branch7x7_1__var, branch7x7_2__w, branch7x7_2__gamma, branch7x7_2__beta, branch7x7_2__mean, branch7x7_2__var, branch7x7_3__w, branch7x7_3__gamma, branch7x7_3__beta, branch7x7_3__mean, branch7x7_3__var, branch7x7dbl_1__w, branch7x7dbl_1__gamma, branch7x7dbl_1__beta, branch7x7dbl_1__mean, branch7x7dbl_1__var, branch7x7dbl_2__w, branch7x7dbl_2__gamma, branch7x7dbl_2__beta, branch7x7dbl_2__mean, branch7x7dbl_2__var, branch7x7dbl_3__w, branch7x7dbl_3__gamma, branch7x7dbl_3__beta, branch7x7dbl_3__mean, branch7x7dbl_3__var, branch7x7dbl_4__w, branch7x7dbl_4__gamma, branch7x7dbl_4__beta, branch7x7dbl_4__mean, branch7x7dbl_4__var, branch7x7dbl_5__w, branch7x7dbl_5__gamma, branch7x7dbl_5__beta, branch7x7dbl_5__mean, branch7x7dbl_5__var, branch_pool__w, branch_pool__gamma, branch_pool__beta, branch_pool__mean, branch_pool__var):
    params = {
        "branch1x1": {"w": branch1x1__w, "gamma": branch1x1__gamma, "beta": branch1x1__beta, "mean": branch1x1__mean, "var": branch1x1__var},
        "branch7x7_1": {"w": branch7x7_1__w, "gamma": branch7x7_1__gamma, "beta": branch7x7_1__beta, "mean": branch7x7_1__mean, "var": branch7x7_1__var},
        "branch7x7_2": {"w": branch7x7_2__w, "gamma": branch7x7_2__gamma, "beta": branch7x7_2__beta, "mean": branch7x7_2__mean, "var": branch7x7_2__var},
        "branch7x7_3": {"w": branch7x7_3__w, "gamma": branch7x7_3__gamma, "beta": branch7x7_3__beta, "mean": branch7x7_3__mean, "var": branch7x7_3__var},
        "branch7x7dbl_1": {"w": branch7x7dbl_1__w, "gamma": branch7x7dbl_1__gamma, "beta": branch7x7dbl_1__beta, "mean": branch7x7dbl_1__mean, "var": branch7x7dbl_1__var},
        "branch7x7dbl_2": {"w": branch7x7dbl_2__w, "gamma": branch7x7dbl_2__gamma, "beta": branch7x7dbl_2__beta, "mean": branch7x7dbl_2__mean, "var": branch7x7dbl_2__var},
        "branch7x7dbl_3": {"w": branch7x7dbl_3__w, "gamma": branch7x7dbl_3__gamma, "beta": branch7x7dbl_3__beta, "mean": branch7x7dbl_3__mean, "var": branch7x7dbl_3__var},
        "branch7x7dbl_4": {"w": branch7x7dbl_4__w, "gamma": branch7x7dbl_4__gamma, "beta": branch7x7dbl_4__beta, "mean": branch7x7dbl_4__mean, "var": branch7x7dbl_4__var},
        "branch7x7dbl_5": {"w": branch7x7dbl_5__w, "gamma": branch7x7dbl_5__gamma, "beta": branch7x7dbl_5__beta, "mean": branch7x7dbl_5__mean, "var": branch7x7dbl_5__var},
        "branch_pool": {"w": branch_pool__w, "gamma": branch_pool__gamma, "beta": branch_pool__beta, "mean": branch_pool__mean, "var": branch_pool__var},
    }
    return _forward(x, params)


# speedup vs baseline: 1.3790x; 1.2050x over previous
import functools

import numpy as np

import jax
import jax.numpy as jnp
from jax import lax
from jax.experimental import pallas as pl
from jax.experimental.pallas import tpu as pltpu

_BN_EPS = 1e-3
_VMEM_LIMIT = 48 * 1024 * 1024
_G = 2


def _round_up(x, m):
    return ((x + m - 1) // m) * m


def _fold_bn(p):
    scale = p["gamma"] * lax.rsqrt(p["var"] + _BN_EPS)
    bias = p["beta"] - p["mean"] * scale
    return scale, bias


def _pw_weight(p):
    scale, bias = _fold_bn(p)
    return p["w"][:, :, 0, 0] * scale[:, None], bias


def _tap_weight(p, axis):
    scale, bias = _fold_bn(p)
    w = p["w"][:, :, 0, :] if axis == "w" else p["w"][:, :, :, 0]
    w = jnp.transpose(w, (0, 2, 1)) * scale[:, None, None]
    return w.reshape(w.shape[0], -1), bias


def _offset_masks(h, w, lp, offsets, g):
    l = np.arange(lp)
    hh, ww = l // w, l % w
    valid = l < h * w
    rows = [valid & (hh + dh >= 0) & (hh + dh < h) & (ww + dw >= 0) & (ww + dw < w)
            for dh, dw in offsets]
    m = np.stack(rows).astype(np.float32)
    return jnp.asarray(np.tile(m, (1, g)))


def _conv7(h, xs_ref, w_ref, b_ref, m_ref, unit, n2):
    c = h.shape[0]
    for k in range(7):
        shift = ((3 - k) * unit) % n2
        xk = pltpu.roll(h, shift=shift, axis=1) if shift else h
        xs_ref[k * c:(k + 1) * c, :] = xk * m_ref[k:k + 1, :]
    y = jnp.dot(w_ref[...], xs_ref[...], preferred_element_type=jnp.float32)
    return jnp.maximum(y + b_ref[...], 0.0)


def _fused_kernel(w0_ref, b0_ref, pb_ref, mw_ref, mh_ref, mv_ref, mp_ref,
                  w2_ref, b2_ref, w3_ref, b3_ref,
                  wd2_ref, bd2_ref, wd3_ref, bd3_ref,
                  wd4_ref, bd4_ref, wd5_ref, bd5_ref,
                  x_ref, o_ref, xs_ref, xa_ref, xb_ref,
                  *, c7, w_stride, l, lp, g):
    n2 = g * lp
    bf = jnp.bfloat16
    for i in range(g):
        xs_ref[:, i * lp:i * lp + l] = x_ref[i].astype(bf)
        if l < lp:
            xs_ref[:, i * lp + l:(i + 1) * lp] = jnp.zeros(
                (x_ref.shape[1], lp - l), bf)
    x2 = xs_ref[...]

    y0 = jnp.dot(w0_ref[...], x2, preferred_element_type=jnp.float32)
    y0 = y0 + b0_ref[...]
    h7 = jnp.maximum(y0[:c7], 0.0).astype(bf)
    hd = jnp.maximum(y0[c7:2 * c7], 0.0).astype(bf)
    b1 = jnp.maximum(y0[2 * c7:2 * c7 + 192], 0.0)
    pz = y0[2 * c7 + 192:]

    t7 = _conv7(h7, xa_ref, w2_ref, b2_ref, mw_ref, 1, n2).astype(bf)
    td = _conv7(hd, xb_ref, wd2_ref, bd2_ref, mh_ref, w_stride, n2).astype(bf)
    b7 = _conv7(t7, xa_ref, w3_ref, b3_ref, mh_ref, w_stride, n2)
    td = _conv7(td, xb_ref, wd3_ref, bd3_ref, mw_ref, 1, n2).astype(bf)
    td = _conv7(td, xb_ref, wd4_ref, bd4_ref, mh_ref, w_stride, n2).astype(bf)
    bd = _conv7(td, xb_ref, wd5_ref, bd5_ref, mw_ref, 1, n2)

    pv = None
    for i, dh in enumerate((-1, 0, 1)):
        shift = (-dh * w_stride) % n2
        part = pltpu.roll(pz, shift=shift, axis=1) if shift else pz
        term = part * mv_ref[i:i + 1, :]
        pv = term if pv is None else pv + term
    pp = None
    for i, dw in enumerate((-1, 0, 1)):
        shift = (-dw) % n2
        part = pltpu.roll(pv, shift=shift, axis=1) if shift else pv
        term = part * mp_ref[i:i + 1, :]
        pp = term if pp is None else pp + term
    bp = jnp.maximum(pp + pb_ref[...], 0.0)

    out = jnp.concatenate([b1, b7, bd, bp], axis=0)
    for i in range(g):
        o_ref[i] = out[:, i * lp:i * lp + l].astype(o_ref.dtype)


@jax.jit
def _forward(x, params):
    n, c_in, h, w = x.shape
    c7 = params["branch7x7_1"]["w"].shape[0]
    l = h * w
    lp = _round_up(l, 128)
    g = _G if n % _G == 0 else 1
    n2 = g * lp
    bf = jnp.bfloat16

    xf = x.reshape(n, c_in, l).astype(bf)

    w_h7, b_h7 = _pw_weight(params["branch7x7_1"])
    w_hd, b_hd = _pw_weight(params["branch7x7dbl_1"])
    w_b1, b_b1 = _pw_weight(params["branch1x1"])
    scale_p, bias_p = _fold_bn(params["branch_pool"])
    w_bp = params["branch_pool"]["w"][:, :, 0, 0] * (scale_p[:, None] / 9.0)
    w0 = jnp.concatenate([w_h7, w_hd, w_b1, w_bp], axis=0).astype(bf)
    b0 = jnp.concatenate(
        [b_h7, b_hd, b_b1, jnp.zeros_like(bias_p)], axis=0)[:, None]

    w2, b2 = _tap_weight(params["branch7x7_2"], "w")
    w3, b3 = _tap_weight(params["branch7x7_3"], "h")
    wd2, bd2 = _tap_weight(params["branch7x7dbl_2"], "h")
    wd3, bd3 = _tap_weight(params["branch7x7dbl_3"], "w")
    wd4, bd4 = _tap_weight(params["branch7x7dbl_4"], "h")
    wd5, bd5 = _tap_weight(params["branch7x7dbl_5"], "w")

    mask_w = _offset_masks(h, w, lp, [(0, k - 3) for k in range(7)], g).astype(bf)
    mask_h = _offset_masks(h, w, lp, [(k - 3, 0) for k in range(7)], g).astype(bf)
    mask_v = _offset_masks(h, w, lp, [(dh, 0) for dh in (-1, 0, 1)], g)
    mask_p = _offset_masks(h, w, lp, [(0, dw) for dw in (-1, 0, 1)], g)

    m_tot = w0.shape[0]
    c_out = 4 * 192

    def bcast(shape):
        return pl.BlockSpec(shape, lambda i: tuple(0 for _ in shape))

    out = pl.pallas_call(
        functools.partial(_fused_kernel, c7=c7, w_stride=w, l=l, lp=lp, g=g),
        out_shape=jax.ShapeDtypeStruct((n, c_out, l), bf),
        grid=(n // g,),
        in_specs=[
            bcast((m_tot, c_in)),
            bcast((m_tot, 1)),
            bcast((192, 1)),
            bcast((7, n2)),
            bcast((7, n2)),
            bcast((3, n2)),
            bcast((3, n2)),
            bcast((c7, 7 * c7)), bcast((c7, 1)),
            bcast((192, 7 * c7)), bcast((192, 1)),
            bcast((c7, 7 * c7)), bcast((c7, 1)),
            bcast((c7, 7 * c7)), bcast((c7, 1)),
            bcast((c7, 7 * c7)), bcast((c7, 1)),
            bcast((192, 7 * c7)), bcast((192, 1)),
            pl.BlockSpec((g, c_in, l), lambda i: (i, 0, 0)),
        ],
        out_specs=pl.BlockSpec((g, c_out, l), lambda i: (i, 0, 0)),
        scratch_shapes=[pltpu.VMEM((c_in, g * lp), bf),
                        pltpu.VMEM((7 * c7, g * lp), bf),
                        pltpu.VMEM((7 * c7, g * lp), bf)],
        compiler_params=pltpu.CompilerParams(
            dimension_semantics=("parallel",),
            vmem_limit_bytes=_VMEM_LIMIT),
    )(w0, b0, bias_p[:, None],
      mask_w, mask_h, mask_v, mask_p,
      w2.astype(bf), b2[:, None], w3.astype(bf), b3[:, None],
      wd2.astype(bf), bd2[:, None], wd3.astype(bf), bd3[:, None],
      wd4.astype(bf), bd4[:, None], wd5.astype(bf), bd5[:, None],
      xf)

    return out.astype(jnp.float32).reshape(n, c_out, h, w)


def kernel(x, branch1x1__w, branch1x1__gamma, branch1x1__beta, branch1x1__mean, branch1x1__var, branch7x7_1__w, branch7x7_1__gamma, branch7x7_1__beta, branch7x7_1__mean, branch7x7_1__var, branch7x7_2__w, branch7x7_2__gamma, branch7x7_2__beta, branch7x7_2__mean, branch7x7_2__var, branch7x7_3__w, branch7x7_3__gamma, branch7x7_3__beta, branch7x7_3__mean, branch7x7_3__var, branch7x7dbl_1__w, branch7x7dbl_1__gamma, branch7x7dbl_1__beta, branch7x7dbl_1__mean, branch7x7dbl_1__var, branch7x7dbl_2__w, branch7x7dbl_2__gamma, branch7x7dbl_2__beta, branch7x7dbl_2__mean, branch7x7dbl_2__var, branch7x7dbl_3__w, branch7x7dbl_3__gamma, branch7x7dbl_3__beta, branch7x7dbl_3__mean, branch7x7dbl_3__var, branch7x7dbl_4__w, branch7x7dbl_4__gamma, branch7x7dbl_4__beta, branch7x7dbl_4__mean, branch7x7dbl_4__var, branch7x7dbl_5__w, branch7x7dbl_5__gamma, branch7x7dbl_5__beta, branch7x7dbl_5__mean, branch7x7dbl_5__var, branch_pool__w, branch_pool__gamma, branch_pool__beta, branch_pool__mean, branch_pool__var):
    params = {
        "branch1x1": {"w": branch1x1__w, "gamma": branch1x1__gamma, "beta": branch1x1__beta, "mean": branch1x1__mean, "var": branch1x1__var},
        "branch7x7_1": {"w": branch7x7_1__w, "gamma": branch7x7_1__gamma, "beta": branch7x7_1__beta, "mean": branch7x7_1__mean, "var": branch7x7_1__var},
        "branch7x7_2": {"w": branch7x7_2__w, "gamma": branch7x7_2__gamma, "beta": branch7x7_2__beta, "mean": branch7x7_2__mean, "var": branch7x7_2__var},
        "branch7x7_3": {"w": branch7x7_3__w, "gamma": branch7x7_3__gamma, "beta": branch7x7_3__beta, "mean": branch7x7_3__mean, "var": branch7x7_3__var},
        "branch7x7dbl_1": {"w": branch7x7dbl_1__w, "gamma": branch7x7dbl_1__gamma, "beta": branch7x7dbl_1__beta, "mean": branch7x7dbl_1__mean, "var": branch7x7dbl_1__var},
        "branch7x7dbl_2": {"w": branch7x7dbl_2__w, "gamma": branch7x7dbl_2__gamma, "beta": branch7x7dbl_2__beta, "mean": branch7x7dbl_2__mean, "var": branch7x7dbl_2__var},
        "branch7x7dbl_3": {"w": branch7x7dbl_3__w, "gamma": branch7x7dbl_3__gamma, "beta": branch7x7dbl_3__beta, "mean": branch7x7dbl_3__mean, "var": branch7x7dbl_3__var},
        "branch7x7dbl_4": {"w": branch7x7dbl_4__w, "gamma": branch7x7dbl_4__gamma, "beta": branch7x7dbl_4__beta, "mean": branch7x7dbl_4__mean, "var": branch7x7dbl_4__var},
        "branch7x7dbl_5": {"w": branch7x7dbl_5__w, "gamma": branch7x7dbl_5__gamma, "beta": branch7x7dbl_5__beta, "mean": branch7x7dbl_5__mean, "var": branch7x7dbl_5__var},
        "branch_pool": {"w": branch_pool__w, "gamma": branch_pool__gamma, "beta": branch_pool__beta, "mean": branch_pool__mean, "var": branch_pool__var},
    }
    return _forward(x, params)


# speedup vs baseline: 1.5749x; 1.1420x over previous
import functools

import numpy as np

import jax
import jax.numpy as jnp
from jax import lax
from jax.experimental import pallas as pl
from jax.experimental.pallas import tpu as pltpu

_BN_EPS = 1e-3
_VMEM_LIMIT = 48 * 1024 * 1024
_G = 4


def _round_up(x, m):
    return ((x + m - 1) // m) * m


def _fold_bn(p):
    scale = p["gamma"] * lax.rsqrt(p["var"] + _BN_EPS)
    bias = p["beta"] - p["mean"] * scale
    return scale, bias


def _pw_weight(p):
    scale, bias = _fold_bn(p)
    return p["w"][:, :, 0, 0] * scale[:, None], bias


def _tap_weight(p, axis):
    scale, bias = _fold_bn(p)
    w = p["w"][:, :, 0, :] if axis == "w" else p["w"][:, :, :, 0]
    w = jnp.transpose(w, (0, 2, 1)) * scale[:, None, None]
    return w.reshape(w.shape[0], -1), bias


def _offset_masks(h, w, lp, offsets, g):
    l = np.arange(lp)
    hh, ww = l // w, l % w
    valid = l < h * w
    rows = [valid & (hh + dh >= 0) & (hh + dh < h) & (ww + dw >= 0) & (ww + dw < w)
            for dh, dw in offsets]
    m = np.stack(rows).astype(np.float32)
    return jnp.asarray(np.tile(m, (1, g)))


def _conv7(h, xs_ref, w_ref, b_ref, m_ref, unit, n2):
    c = h.shape[0]
    for k in range(7):
        shift = ((3 - k) * unit) % n2
        xk = pltpu.roll(h, shift=shift, axis=1) if shift else h
        xs_ref[k * c:(k + 1) * c, :] = xk * m_ref[k:k + 1, :]
    y = jnp.dot(w_ref[...], xs_ref[...], preferred_element_type=jnp.float32)
    return jnp.maximum(y + b_ref[...], 0.0)


def _fused_kernel(w0_ref, b0_ref, pb_ref, mw_ref, mh_ref, mv_ref, mp_ref,
                  w2_ref, b2_ref, w3_ref, b3_ref,
                  wd2_ref, bd2_ref, wd3_ref, bd3_ref,
                  wd4_ref, bd4_ref, wd5_ref, bd5_ref,
                  x_ref, o_ref, xs_ref, xa_ref, xb_ref,
                  *, c7, w_stride, l, lp, g):
    n2 = g * lp
    bf = jnp.bfloat16
    for i in range(g):
        xs_ref[:, i * lp:i * lp + l] = x_ref[i].astype(bf)
        if l < lp:
            xs_ref[:, i * lp + l:(i + 1) * lp] = jnp.zeros(
                (x_ref.shape[1], lp - l), bf)
    x2 = xs_ref[...]

    y0 = jnp.dot(w0_ref[...], x2, preferred_element_type=jnp.float32)
    y0 = y0 + b0_ref[...]
    h7 = jnp.maximum(y0[:c7], 0.0).astype(bf)
    hd = jnp.maximum(y0[c7:2 * c7], 0.0).astype(bf)
    b1 = jnp.maximum(y0[2 * c7:2 * c7 + 192], 0.0)
    pz = y0[2 * c7 + 192:].astype(bf)

    t7 = _conv7(h7, xa_ref, w2_ref, b2_ref, mw_ref, 1, n2).astype(bf)
    td = _conv7(hd, xb_ref, wd2_ref, bd2_ref, mh_ref, w_stride, n2).astype(bf)
    b7 = _conv7(t7, xa_ref, w3_ref, b3_ref, mh_ref, w_stride, n2)
    td = _conv7(td, xb_ref, wd3_ref, bd3_ref, mw_ref, 1, n2).astype(bf)
    td = _conv7(td, xb_ref, wd4_ref, bd4_ref, mh_ref, w_stride, n2).astype(bf)
    bd = _conv7(td, xb_ref, wd5_ref, bd5_ref, mw_ref, 1, n2)

    pv = None
    for i, dh in enumerate((-1, 0, 1)):
        shift = (-dh * w_stride) % n2
        part = pltpu.roll(pz, shift=shift, axis=1) if shift else pz
        term = part * mv_ref[i:i + 1, :]
        pv = term if pv is None else pv + term
    pp = None
    for i, dw in enumerate((-1, 0, 1)):
        shift = (-dw) % n2
        part = pltpu.roll(pv, shift=shift, axis=1) if shift else pv
        term = part * mp_ref[i:i + 1, :]
        pp = term if pp is None else pp + term
    bp = jnp.maximum(pp.astype(jnp.float32) + pb_ref[...], 0.0)

    out = jnp.concatenate([b1, b7, bd, bp], axis=0)
    for i in range(g):
        o_ref[i] = out[:, i * lp:i * lp + l].astype(o_ref.dtype)


@jax.jit
def _forward(x, params):
    n, c_in, h, w = x.shape
    c7 = params["branch7x7_1"]["w"].shape[0]
    l = h * w
    lp = _round_up(l, 128)
    g = _G if n % _G == 0 else 1
    n2 = g * lp
    bf = jnp.bfloat16

    xf = x.reshape(n, c_in, l).astype(bf)

    w_h7, b_h7 = _pw_weight(params["branch7x7_1"])
    w_hd, b_hd = _pw_weight(params["branch7x7dbl_1"])
    w_b1, b_b1 = _pw_weight(params["branch1x1"])
    scale_p, bias_p = _fold_bn(params["branch_pool"])
    w_bp = params["branch_pool"]["w"][:, :, 0, 0] * (scale_p[:, None] / 9.0)
    w0 = jnp.concatenate([w_h7, w_hd, w_b1, w_bp], axis=0).astype(bf)
    b0 = jnp.concatenate(
        [b_h7, b_hd, b_b1, jnp.zeros_like(bias_p)], axis=0)[:, None]

    w2, b2 = _tap_weight(params["branch7x7_2"], "w")
    w3, b3 = _tap_weight(params["branch7x7_3"], "h")
    wd2, bd2 = _tap_weight(params["branch7x7dbl_2"], "h")
    wd3, bd3 = _tap_weight(params["branch7x7dbl_3"], "w")
    wd4, bd4 = _tap_weight(params["branch7x7dbl_4"], "h")
    wd5, bd5 = _tap_weight(params["branch7x7dbl_5"], "w")

    mask_w = _offset_masks(h, w, lp, [(0, k - 3) for k in range(7)], g).astype(bf)
    mask_h = _offset_masks(h, w, lp, [(k - 3, 0) for k in range(7)], g).astype(bf)
    mask_v = _offset_masks(h, w, lp, [(dh, 0) for dh in (-1, 0, 1)], g).astype(bf)
    mask_p = _offset_masks(h, w, lp, [(0, dw) for dw in (-1, 0, 1)], g).astype(bf)

    m_tot = w0.shape[0]
    c_out = 4 * 192

    def bcast(shape):
        return pl.BlockSpec(shape, lambda i: tuple(0 for _ in shape))

    out = pl.pallas_call(
        functools.partial(_fused_kernel, c7=c7, w_stride=w, l=l, lp=lp, g=g),
        out_shape=jax.ShapeDtypeStruct((n, c_out, l), bf),
        grid=(n // g,),
        in_specs=[
            bcast((m_tot, c_in)),
            bcast((m_tot, 1)),
            bcast((192, 1)),
            bcast((7, n2)),
            bcast((7, n2)),
            bcast((3, n2)),
            bcast((3, n2)),
            bcast((c7, 7 * c7)), bcast((c7, 1)),
            bcast((192, 7 * c7)), bcast((192, 1)),
            bcast((c7, 7 * c7)), bcast((c7, 1)),
            bcast((c7, 7 * c7)), bcast((c7, 1)),
            bcast((c7, 7 * c7)), bcast((c7, 1)),
            bcast((192, 7 * c7)), bcast((192, 1)),
            pl.BlockSpec((g, c_in, l), lambda i: (i, 0, 0)),
        ],
        out_specs=pl.BlockSpec((g, c_out, l), lambda i: (i, 0, 0)),
        scratch_shapes=[pltpu.VMEM((c_in, g * lp), bf),
                        pltpu.VMEM((7 * c7, g * lp), bf),
                        pltpu.VMEM((7 * c7, g * lp), bf)],
        compiler_params=pltpu.CompilerParams(
            dimension_semantics=("parallel",),
            vmem_limit_bytes=_VMEM_LIMIT),
    )(w0, b0, bias_p[:, None],
      mask_w, mask_h, mask_v, mask_p,
      w2.astype(bf), b2[:, None], w3.astype(bf), b3[:, None],
      wd2.astype(bf), bd2[:, None], wd3.astype(bf), bd3[:, None],
      wd4.astype(bf), bd4[:, None], wd5.astype(bf), bd5[:, None],
      xf)

    return out.astype(jnp.float32).reshape(n, c_out, h, w)


def kernel(x, branch1x1__w, branch1x1__gamma, branch1x1__beta, branch1x1__mean, branch1x1__var, branch7x7_1__w, branch7x7_1__gamma, branch7x7_1__beta, branch7x7_1__mean, branch7x7_1__var, branch7x7_2__w, branch7x7_2__gamma, branch7x7_2__beta, branch7x7_2__mean, branch7x7_2__var, branch7x7_3__w, branch7x7_3__gamma, branch7x7_3__beta, branch7x7_3__mean, branch7x7_3__var, branch7x7dbl_1__w, branch7x7dbl_1__gamma, branch7x7dbl_1__beta, branch7x7dbl_1__mean, branch7x7dbl_1__var, branch7x7dbl_2__w, branch7x7dbl_2__gamma, branch7x7dbl_2__beta, branch7x7dbl_2__mean, branch7x7dbl_2__var, branch7x7dbl_3__w, branch7x7dbl_3__gamma, branch7x7dbl_3__beta, branch7x7dbl_3__mean, branch7x7dbl_3__var, branch7x7dbl_4__w, branch7x7dbl_4__gamma, branch7x7dbl_4__beta, branch7x7dbl_4__mean, branch7x7dbl_4__var, branch7x7dbl_5__w, branch7x7dbl_5__gamma, branch7x7dbl_5__beta, branch7x7dbl_5__mean, branch7x7dbl_5__var, branch_pool__w, branch_pool__gamma, branch_pool__beta, branch_pool__mean, branch_pool__var):
    params = {
        "branch1x1": {"w": branch1x1__w, "gamma": branch1x1__gamma, "beta": branch1x1__beta, "mean": branch1x1__mean, "var": branch1x1__var},
        "branch7x7_1": {"w": branch7x7_1__w, "gamma": branch7x7_1__gamma, "beta": branch7x7_1__beta, "mean": branch7x7_1__mean, "var": branch7x7_1__var},
        "branch7x7_2": {"w": branch7x7_2__w, "gamma": branch7x7_2__gamma, "beta": branch7x7_2__beta, "mean": branch7x7_2__mean, "var": branch7x7_2__var},
        "branch7x7_3": {"w": branch7x7_3__w, "gamma": branch7x7_3__gamma, "beta": branch7x7_3__beta, "mean": branch7x7_3__mean, "var": branch7x7_3__var},
        "branch7x7dbl_1": {"w": branch7x7dbl_1__w, "gamma": branch7x7dbl_1__gamma, "beta": branch7x7dbl_1__beta, "mean": branch7x7dbl_1__mean, "var": branch7x7dbl_1__var},
        "branch7x7dbl_2": {"w": branch7x7dbl_2__w, "gamma": branch7x7dbl_2__gamma, "beta": branch7x7dbl_2__beta, "mean": branch7x7dbl_2__mean, "var": branch7x7dbl_2__var},
        "branch7x7dbl_3": {"w": branch7x7dbl_3__w, "gamma": branch7x7dbl_3__gamma, "beta": branch7x7dbl_3__beta, "mean": branch7x7dbl_3__mean, "var": branch7x7dbl_3__var},
        "branch7x7dbl_4": {"w": branch7x7dbl_4__w, "gamma": branch7x7dbl_4__gamma, "beta": branch7x7dbl_4__beta, "mean": branch7x7dbl_4__mean, "var": branch7x7dbl_4__var},
        "branch7x7dbl_5": {"w": branch7x7dbl_5__w, "gamma": branch7x7dbl_5__gamma, "beta": branch7x7dbl_5__beta, "mean": branch7x7dbl_5__mean, "var": branch7x7dbl_5__var},
        "branch_pool": {"w": branch_pool__w, "gamma": branch_pool__gamma, "beta": branch_pool__beta, "mean": branch_pool__mean, "var": branch_pool__var},
    }
    return _forward(x, params)
